# baseline (speedup 1.0000x reference)
import functools

import jax
import jax.numpy as jnp
from jax.experimental import pallas as pl
from jax.experimental.pallas import tpu as pltpu


def _proj_block(x_ref, w_ref, b1_ref, out_ref):
    acc = jnp.dot(x_ref[...], w_ref[...], preferred_element_type=jnp.float32)
    out_ref[...] = acc + b1_ref[...]


def _edge_mlp_block(ga_ref, gb_ref, w2_ref, b2_ref, out_ref):
    h = jnp.maximum(ga_ref[...] + gb_ref[...], 0.0)
    out_ref[...] = (
        jnp.dot(h, w2_ref[...], preferred_element_type=jnp.float32) + b2_ref[...]
    )


def kernel(x, edge_index, W1, b1, W2, b2):
    N, C = x.shape
    E = edge_index.shape[1]
    src = edge_index[0]
    dst = edge_index[1]

    A = W1[:C] - W1[C:]
    B = W1[C:]
    AB = jnp.concatenate([A, B], axis=1)
    b1z = jnp.concatenate([b1, jnp.zeros_like(b1)])[None, :]
    BN = 1000
    xAB = pl.pallas_call(
        _proj_block,
        grid=(N // BN,),
        in_specs=[
            pl.BlockSpec((BN, C), lambda i: (i, 0)),
            pl.BlockSpec((C, 2 * C), lambda i: (0, 0)),
            pl.BlockSpec((1, 2 * C), lambda i: (0, 0)),
        ],
        out_specs=pl.BlockSpec((BN, 2 * C), lambda i: (i, 0)),
        out_shape=jax.ShapeDtypeStruct((N, 2 * C), jnp.float32),
    )(x, AB, b1z)
    xA = xAB[:, :C]
    xB = xAB[:, C:]

    ga = jnp.take(xA, dst, axis=0)
    gb = jnp.take(xB, src, axis=0)

    BE = 640
    m = pl.pallas_call(
        _edge_mlp_block,
        grid=(E // BE,),
        in_specs=[
            pl.BlockSpec((BE, C), lambda i: (i, 0)),
            pl.BlockSpec((BE, C), lambda i: (i, 0)),
            pl.BlockSpec((C, C), lambda i: (0, 0)),
            pl.BlockSpec((1, C), lambda i: (0, 0)),
        ],
        out_specs=pl.BlockSpec((BE, C), lambda i: (i, 0)),
        out_shape=jax.ShapeDtypeStruct((E, C), jnp.float32),
    )(ga, gb, W2, b2[None, :])

    seg_max = jax.ops.segment_max(m, dst, num_segments=N)
    counts = jnp.zeros((N,), dtype=m.dtype).at[dst].add(1.0)
    return jnp.where(counts[:, None] > 0, seg_max, 0.0)


# speedup vs baseline: 1.8732x; 1.8732x over previous
import functools

import jax
import jax.numpy as jnp
from jax import lax
from jax.experimental import pallas as pl
from jax.experimental.pallas import tpu as pltpu
from jax.experimental.pallas import tpu_sc as plsc

_NC, _NS = 2, 16
_NW = _NC * _NS


def _sc_gather_body(epw, ch, xa_hbm, xb_hbm, dst_hbm, src_hbm, ga_hbm, gb_hbm,
                    idx_a, idx_b, rows_a, rows_b, sem_a, sem_b):
    wid = lax.axis_index("s") * _NC + lax.axis_index("c")
    base = wid * epw
    nchunk = epw // ch

    def chunk(c, carry):
        off = base + c * ch
        pltpu.sync_copy(dst_hbm.at[pl.ds(off, ch)], idx_a)
        pltpu.sync_copy(src_hbm.at[pl.ds(off, ch)], idx_b)
        cpa = pltpu.async_copy(xa_hbm.at[idx_a], rows_a, sem_a)
        cpb = pltpu.async_copy(xb_hbm.at[idx_b], rows_b, sem_b)
        cpa.wait()
        cpb.wait()
        pltpu.sync_copy(rows_a, ga_hbm.at[pl.ds(off, ch)])
        pltpu.sync_copy(rows_b, gb_hbm.at[pl.ds(off, ch)])
        return carry

    lax.fori_loop(0, nchunk, chunk, 0)


def _sc_gather(xa, xb, dst, src):
    n, c = xa.shape
    e = dst.shape[0]
    epw = e // _NW
    ch = 400
    assert epw % ch == 0 and (epw % 8) == 0
    mesh = plsc.VectorSubcoreMesh(core_axis_name="c", subcore_axis_name="s",
                                  num_cores=_NC, num_subcores=_NS)
    return pl.kernel(
        functools.partial(_sc_gather_body, epw, ch),
        out_type=(jax.ShapeDtypeStruct((e, c), jnp.float32),
                  jax.ShapeDtypeStruct((e, c), jnp.float32)),
        mesh=mesh,
        scratch_types=[
            pltpu.VMEM((ch,), jnp.int32),
            pltpu.VMEM((ch,), jnp.int32),
            pltpu.VMEM((ch, c), jnp.float32),
            pltpu.VMEM((ch, c), jnp.float32),
            pltpu.SemaphoreType.DMA,
            pltpu.SemaphoreType.DMA,
        ],
    )(xa, xb, dst, src)


def _proj_block(x_ref, w_ref, b1_ref, out_ref):
    acc = jnp.dot(x_ref[...], w_ref[...], preferred_element_type=jnp.float32)
    out_ref[...] = acc + b1_ref[...]


def _edge_mlp_block(ga_ref, gb_ref, w2_ref, b2_ref, out_ref):
    h = jnp.maximum(ga_ref[...] + gb_ref[...], 0.0)
    out_ref[...] = (
        jnp.dot(h, w2_ref[...], preferred_element_type=jnp.float32) + b2_ref[...]
    )


def kernel(x, edge_index, W1, b1, W2, b2):
    N, C = x.shape
    E = edge_index.shape[1]
    src = edge_index[0]
    dst = edge_index[1]

    A = W1[:C] - W1[C:]
    B = W1[C:]
    AB = jnp.concatenate([A, B], axis=1)
    b1z = jnp.concatenate([b1, jnp.zeros_like(b1)])[None, :]
    BN = 1000
    xAB = pl.pallas_call(
        _proj_block,
        grid=(N // BN,),
        in_specs=[
            pl.BlockSpec((BN, C), lambda i: (i, 0)),
            pl.BlockSpec((C, 2 * C), lambda i: (0, 0)),
            pl.BlockSpec((1, 2 * C), lambda i: (0, 0)),
        ],
        out_specs=pl.BlockSpec((BN, 2 * C), lambda i: (i, 0)),
        out_shape=jax.ShapeDtypeStruct((N, 2 * C), jnp.float32),
    )(x, AB, b1z)
    xA = xAB[:, :C]
    xB = xAB[:, C:]

    ga, gb = _sc_gather(xA, xB, dst, src)

    BE = 640
    m = pl.pallas_call(
        _edge_mlp_block,
        grid=(E // BE,),
        in_specs=[
            pl.BlockSpec((BE, C), lambda i: (i, 0)),
            pl.BlockSpec((BE, C), lambda i: (i, 0)),
            pl.BlockSpec((C, C), lambda i: (0, 0)),
            pl.BlockSpec((1, C), lambda i: (0, 0)),
        ],
        out_specs=pl.BlockSpec((BE, C), lambda i: (i, 0)),
        out_shape=jax.ShapeDtypeStruct((E, C), jnp.float32),
    )(ga, gb, W2, b2[None, :])

    seg_max = jax.ops.segment_max(m, dst, num_segments=N)
    counts = jnp.zeros((N,), dtype=m.dtype).at[dst].add(1.0)
    return jnp.where(counts[:, None] > 0, seg_max, 0.0)
